# baseline (speedup 1.0000x reference)
import functools

import jax
import jax.numpy as jnp
from jax import lax
from jax.experimental import pallas as pl
from jax.experimental.pallas import tpu as pltpu
from jax.experimental.pallas import tpu_sc as plsc

D_MODEL = 128
SEQ_LEN = 200
IDX_CHUNK = 100
LANES = 16
NVEC = D_MODEL // LANES
EPS = 1e-5


_GATHER_DNUMS = lax.GatherDimensionNumbers(
    offset_dims=(), collapsed_slice_dims=(0,), start_index_map=(0,))


def _permute(v, idx):
    return lax.gather(
        v, idx[:, None], dimension_numbers=_GATHER_DNUMS, slice_sizes=(1,),
        mode=lax.GatherScatterMode.PROMISE_IN_BOUNDS)


def _lane_sum(v):
    idx = lax.iota(jnp.int32, LANES)
    for d in (8, 4, 2, 1):
        v = v + _permute(v, idx ^ d)
    return v


def _rsqrt(y):
    i = lax.bitcast_convert_type(y, jnp.int32)
    i = jnp.int32(0x5F3759DF) - (i >> 1)
    r = lax.bitcast_convert_type(i, jnp.float32)
    half = y * 0.5
    for _ in range(3):
        r = r * (1.5 - half * r * r)
    return r


def _make_sc_kernel(n_seq):
    info = plsc.get_sparse_core_info()
    nw = info.num_cores * info.num_subcores
    seq_per_w = n_seq // nw
    mesh = plsc.VectorSubcoreMesh(core_axis_name="c", subcore_axis_name="s")

    @functools.partial(
        pl.kernel,
        out_type=jax.ShapeDtypeStruct((n_seq * SEQ_LEN, D_MODEL), jnp.float32),
        mesh=mesh,
        scratch_types=[
            pltpu.VMEM((2, IDX_CHUNK), jnp.int32),
            pltpu.VMEM((SEQ_LEN, D_MODEL), jnp.float32),
            pltpu.VMEM((SEQ_LEN, D_MODEL), jnp.float32),
            pltpu.VMEM((D_MODEL,), jnp.float32),
            pltpu.VMEM((D_MODEL,), jnp.float32),
            pltpu.SemaphoreType.DMA,
        ],
    )
    def sc_kernel(x_hbm, table_hbm, pe_hbm, gamma_hbm, beta_hbm, out_hbm,
                  idx_v, emb_v, pe_v, g_v, b_v, sem):
        wid = lax.axis_index("s") * info.num_cores + lax.axis_index("c")

        pltpu.sync_copy(pe_hbm, pe_v)
        pltpu.sync_copy(gamma_hbm, g_v)
        pltpu.sync_copy(beta_hbm, b_v)

        def per_seq(s, carry):
            seq = wid * seq_per_w + s
            pltpu.sync_copy(x_hbm.at[seq], idx_v)
            cp0 = pltpu.async_copy(
                table_hbm.at[idx_v.at[0]], emb_v.at[pl.ds(0, IDX_CHUNK)], sem)
            cp1 = pltpu.async_copy(
                table_hbm.at[idx_v.at[1]],
                emb_v.at[pl.ds(IDX_CHUNK, IDX_CHUNK)], sem)
            cp0.wait()
            cp1.wait()

            def per_row(r, c):
                v = [emb_v[r, pl.ds(j * LANES, LANES)]
                     + pe_v[r, pl.ds(j * LANES, LANES)]
                     for j in range(NVEC)]
                s1 = (v[0] + v[1]) + (v[2] + v[3])
                s2 = (v[4] + v[5]) + (v[6] + v[7])
                tot = _lane_sum(s1 + s2)
                q1 = (v[0] * v[0] + v[1] * v[1]) + (v[2] * v[2] + v[3] * v[3])
                q2 = (v[4] * v[4] + v[5] * v[5]) + (v[6] * v[6] + v[7] * v[7])
                qtot = _lane_sum(q1 + q2)
                mean = tot * (1.0 / D_MODEL)
                var = qtot * (1.0 / D_MODEL) - mean * mean
                rstd = _rsqrt(var + EPS)
                for j in range(NVEC):
                    g = g_v[pl.ds(j * LANES, LANES)]
                    b = b_v[pl.ds(j * LANES, LANES)]
                    emb_v[r, pl.ds(j * LANES, LANES)] = (
                        (v[j] - mean) * rstd * g + b)
                return c

            lax.fori_loop(0, SEQ_LEN, per_row, 0)
            pltpu.sync_copy(emb_v, out_hbm.at[pl.ds(seq * SEQ_LEN, SEQ_LEN)])
            return carry

        lax.fori_loop(0, seq_per_w, per_seq, 0)

    return sc_kernel


def kernel(x, table, pe, gamma, beta):
    n_seq, seq_len = x.shape
    assert seq_len == SEQ_LEN
    xc = x.astype(jnp.int32).reshape(n_seq, 2, IDX_CHUNK)
    pe2 = pe[0, :SEQ_LEN, :]
    out = _make_sc_kernel(n_seq)(xc, table, pe2, gamma, beta)
    return out.reshape(n_seq, seq_len, D_MODEL)


# speedup vs baseline: 2.7801x; 2.7801x over previous
import functools

import jax
import jax.numpy as jnp
from jax import lax
from jax.experimental import pallas as pl
from jax.experimental.pallas import tpu as pltpu
from jax.experimental.pallas import tpu_sc as plsc

D_MODEL = 128
SEQ_LEN = 200
CHUNK = 128
NBUF = 5
TC_SEQS = 8
EPS = 1e-5


def _make_sc_gather(n_tok):
    info = plsc.get_sparse_core_info()
    nw = info.num_cores * info.num_subcores
    tok_per_w = n_tok // nw
    n_chunks = tok_per_w // CHUNK
    assert n_chunks % NBUF == 0
    mesh = plsc.VectorSubcoreMesh(core_axis_name="c", subcore_axis_name="s")

    @functools.partial(
        pl.kernel,
        out_type=jax.ShapeDtypeStruct((n_tok, D_MODEL), jnp.float32),
        mesh=mesh,
        scratch_types=[
            pltpu.VMEM((n_chunks, CHUNK), jnp.int32),
            pltpu.VMEM((NBUF, CHUNK, D_MODEL), jnp.float32),
            pltpu.SemaphoreType.DMA((NBUF,)),
            pltpu.SemaphoreType.DMA((NBUF,)),
        ],
    )
    def sc_gather(x_hbm, table_hbm, out_hbm, idx_v, bufs, gsem, osem):
        wid = lax.axis_index("s") * info.num_cores + lax.axis_index("c")
        base = wid * tok_per_w
        pltpu.sync_copy(x_hbm.at[wid], idx_v)

        def out_slice(j):
            return out_hbm.at[pl.ds(base + j * CHUNK, CHUNK)]

        def ring_round(g, carry):
            cps = []
            for b in range(NBUF):
                @pl.when(g > 0)
                def _wait_prev():
                    pltpu.make_async_copy(
                        bufs.at[b], out_slice(g - NBUF + b), osem.at[b]
                    ).wait()
                cps.append(pltpu.async_copy(
                    table_hbm.at[idx_v.at[g + b]], bufs.at[b], gsem.at[b]))
            for b in range(NBUF):
                cps[b].wait()
                pltpu.async_copy(bufs.at[b], out_slice(g + b), osem.at[b])
            return carry

        lax.fori_loop(0, n_chunks // NBUF, lambda i, c: ring_round(i * NBUF, c),
                      0, unroll=False)
        for b in range(NBUF):
            pltpu.make_async_copy(
                bufs.at[b], out_slice(n_chunks - NBUF + b), osem.at[b]).wait()

    return sc_gather


def _tc_ln_body(emb_ref, pe_ref, g_ref, b_ref, out_ref):
    e = emb_ref[...] + pe_ref[...]
    mean = jnp.mean(e, axis=-1, keepdims=True)
    c = e - mean
    var = jnp.mean(c * c, axis=-1, keepdims=True)
    out_ref[...] = c * lax.rsqrt(var + EPS) * g_ref[...] + b_ref[...]


def _tc_ln(emb, pe_big, gamma, beta):
    n_tok = emb.shape[0]
    rows = TC_SEQS * SEQ_LEN
    grid = n_tok // rows
    return pl.pallas_call(
        _tc_ln_body,
        grid=(grid,),
        in_specs=[
            pl.BlockSpec((rows, D_MODEL), lambda i: (i, 0)),
            pl.BlockSpec((rows, D_MODEL), lambda i: (0, 0)),
            pl.BlockSpec((1, D_MODEL), lambda i: (0, 0)),
            pl.BlockSpec((1, D_MODEL), lambda i: (0, 0)),
        ],
        out_specs=pl.BlockSpec((rows, D_MODEL), lambda i: (i, 0)),
        out_shape=jax.ShapeDtypeStruct((n_tok, D_MODEL), jnp.float32),
    )(emb, pe_big, gamma, beta)


def kernel(x, table, pe, gamma, beta):
    n_seq, seq_len = x.shape
    assert seq_len == SEQ_LEN
    n_tok = n_seq * seq_len
    info = plsc.get_sparse_core_info()
    nw = info.num_cores * info.num_subcores
    xc = x.astype(jnp.int32).reshape(nw, n_tok // nw // CHUNK, CHUNK)
    gathered = _make_sc_gather(n_tok)(xc, table)
    pe_big = jnp.tile(pe[0, :SEQ_LEN, :], (TC_SEQS, 1))
    out = _tc_ln(gathered, pe_big, gamma.reshape(1, D_MODEL),
                 beta.reshape(1, D_MODEL))
    return out.reshape(n_seq, seq_len, D_MODEL)
